# baseline (speedup 1.0000x reference)
import dataclasses
import functools

import jax
import jax.numpy as jnp
from jax import lax
from jax.experimental import pallas as pl
from jax.experimental.pallas import tpu as pltpu
from jax.experimental.pallas import tpu_sc as plsc

N = 10000
E = 320000
D = 128
NC, NS = 2, 16
NW = NC * NS
FPT = D // NW
BLK = 128
NB_DEG = 79
CH = 1024
NCH = 316
E_PAD = NCH * CH
N_PAD = 10240
TW = FPT * N_PAD
LANES = 16

_mesh = plsc.VectorSubcoreMesh(
    core_axis_name="c", subcore_axis_name="s", num_cores=NC, num_subcores=NS
)

_f32 = jnp.float32

_sc_params = pltpu.CompilerParams()
if "needs_layout_passes" in pltpu.CompilerParams.__dataclass_fields__:
    _sc_params = dataclasses.replace(_sc_params, needs_layout_passes=False)


@functools.partial(
    pl.kernel,
    out_type=jax.ShapeDtypeStruct((NW * N,), _f32),
    mesh=_mesh,
    scratch_types=[
        pltpu.VMEM((NB_DEG, BLK), jnp.int32),
        pltpu.VMEM((NB_DEG, BLK), _f32),
        pltpu.VMEM((N,), _f32),
    ],
    compiler_params=_sc_params,
)
def _deg_kernel(ei_hbm, w_hbm, out_hbm, colb, wb, degloc):
    c = lax.axis_index("c")
    s = lax.axis_index("s")
    wid = c * NS + s
    pltpu.sync_copy(ei_hbm.at[1, wid], colb)
    pltpu.sync_copy(w_hbm.at[wid], wb)

    @pl.loop(0, N, step=LANES)
    def _zero(i):
        degloc[pl.ds(i, LANES)] = jnp.zeros((LANES,), _f32)

    @pl.loop(0, NB_DEG)
    def _blocks(blk):
        @pl.loop(0, BLK, step=LANES)
        def _groups(j0):
            col16 = colb[blk, pl.ds(j0, LANES)]
            w16 = wb[blk, pl.ds(j0, LANES)]
            plsc.addupdate_scatter(degloc, [col16], w16)

    pltpu.sync_copy(degloc, out_hbm.at[pl.ds(wid * N, N)])


@functools.partial(
    pl.kernel,
    out_type=jax.ShapeDtypeStruct((NW, TW), _f32),
    mesh=_mesh,
    scratch_types=[
        pltpu.VMEM((TW,), _f32),
        pltpu.VMEM((TW,), _f32),
        pltpu.VMEM((CH,), jnp.int32),
        pltpu.VMEM((CH,), jnp.int32),
        pltpu.VMEM((CH,), jnp.int32),
        pltpu.VMEM((CH,), jnp.int32),
        pltpu.VMEM((CH,), _f32),
        pltpu.VMEM((CH,), _f32),
        pltpu.SemaphoreType.DMA,
        pltpu.SemaphoreType.DMA,
    ],
    compiler_params=_sc_params,
)
def _agg_kernel(hq_hbm, ei_hbm, w_hbm, out_hbm, table, acc,
                row0, row1, col0, col1, w0, w1, esem0, esem1):
    c = lax.axis_index("c")
    s = lax.axis_index("s")
    wid = c * NS + s

    pltpu.sync_copy(hq_hbm.at[wid], table)

    @pl.loop(0, TW, step=LANES)
    def _zero(i):
        acc[pl.ds(i, LANES)] = jnp.zeros((LANES,), _f32)

    def fetch(k, rowP, colP, wP, esemP):
        pltpu.async_copy(ei_hbm.at[0, k], rowP, esemP)
        pltpu.async_copy(ei_hbm.at[1, k], colP, esemP)
        pltpu.async_copy(w_hbm.at[k], wP, esemP)

    def fetch_wait(k, rowP, colP, wP, esemP):
        pltpu.make_async_copy(ei_hbm.at[0, k], rowP, esemP).wait()
        pltpu.make_async_copy(ei_hbm.at[1, k], colP, esemP).wait()
        pltpu.make_async_copy(w_hbm.at[k], wP, esemP).wait()

    fetch(0, row0, col0, w0, esem0)
    fetch(1, row1, col1, w1, esem1)

    def do_chunk(k, rowP, colP, wP, esemP):
        fetch_wait(k, rowP, colP, wP, esemP)

        @pl.loop(0, CH, step=4 * LANES)
        def _groups(g0):
            for u in range(4):
                j0 = g0 + u * LANES
                row16 = rowP[pl.ds(j0, LANES)]
                col16 = colP[pl.ds(j0, LANES)]
                w16 = wP[pl.ds(j0, LANES)]
                for f in range(FPT):
                    tf = table.at[pl.ds(f * N_PAD, N_PAD)]
                    af = acc.at[pl.ds(f * N_PAD, N_PAD)]
                    v = plsc.load_gather(tf, [row16])
                    plsc.addupdate_scatter(af, [col16], v * w16)

        @pl.when(k + 2 < NCH)
        def _():
            fetch(k + 2, rowP, colP, wP, esemP)

    @pl.loop(0, NCH // 2)
    def _pairs(i):
        k = i * 2
        do_chunk(k, row0, col0, w0, esem0)
        do_chunk(k + 1, row1, col1, w1, esem1)

    pltpu.sync_copy(acc, out_hbm.at[wid])


def _tc1_body(parts_ref, x_ref, w1_ref, h1p_ref, dinv_ref):
    deg = jnp.sum(parts_ref[...], axis=0) + 1.0
    dinv = jnp.where(deg > 0, lax.rsqrt(deg), 0.0)[:, None]
    h1 = jnp.dot(x_ref[...], w1_ref[...], preferred_element_type=_f32)
    h1p_ref[...] = h1 * dinv
    dinv_ref[...] = dinv


def _tc2_body(agg_ref, h1p_ref, dinv_ref, b1_ref, w2_ref, h2p_ref):
    dinv = dinv_ref[...]
    x2 = jnp.maximum((agg_ref[:N] + h1p_ref[...]) * dinv + b1_ref[...], 0.0)
    h2 = jnp.dot(x2, w2_ref[...], preferred_element_type=_f32)
    h2p_ref[...] = h2 * dinv


def _tc3_body(agg_ref, h2p_ref, dinv_ref, b2_ref, bn_g_ref, bn_b_ref,
              ln_g_ref, ln_b_ref, out_ref):
    dinv = dinv_ref[...]
    t = jnp.maximum((agg_ref[:N] + h2p_ref[...]) * dinv + b2_ref[...], 0.0)
    mu = jnp.mean(t, axis=0, keepdims=True)
    var = jnp.mean((t - mu) ** 2, axis=0, keepdims=True)
    h = (t - mu) / jnp.sqrt(var + 1e-5) * bn_g_ref[...] + bn_b_ref[...]
    lmu = jnp.mean(h, axis=1, keepdims=True)
    lvar = jnp.mean((h - lmu) ** 2, axis=1, keepdims=True)
    out_ref[...] = (h - lmu) / jnp.sqrt(lvar + 1e-5) * ln_g_ref[...] + ln_b_ref[...]


def _to_feature_major(hp):
    hp_pad = jnp.concatenate(
        [hp, jnp.zeros((N_PAD - N, D), hp.dtype)], axis=0
    )
    return hp_pad.T.reshape(NW, FPT * N_PAD)


def _from_feature_major(o):
    return o.reshape(D, N_PAD).T


def kernel(x, edge_index, edge_weight, W1, b1, W2, b2, bn_g, bn_b, ln_g, ln_b):
    pad = E_PAD - E
    ei_flat = jnp.concatenate(
        [edge_index, jnp.zeros((2, pad), edge_index.dtype)], axis=1
    )
    w_flat = jnp.concatenate([edge_weight, jnp.zeros((pad,), edge_weight.dtype)])
    ei_deg = ei_flat.reshape(2, NW, NB_DEG, BLK)
    w_deg = w_flat.reshape(NW, NB_DEG, BLK)
    ei_agg = ei_flat.reshape(2, NCH, CH)
    w_agg = w_flat.reshape(NCH, CH)

    parts = _deg_kernel(ei_deg, w_deg).reshape(NW, N)

    h1p, dinv = pl.pallas_call(
        _tc1_body,
        out_shape=[jax.ShapeDtypeStruct((N, D), _f32),
                   jax.ShapeDtypeStruct((N, 1), _f32)],
    )(parts, x, W1)

    agg1 = _from_feature_major(_agg_kernel(_to_feature_major(h1p), ei_agg, w_agg))

    h2p = pl.pallas_call(
        _tc2_body,
        out_shape=jax.ShapeDtypeStruct((N, D), _f32),
    )(agg1, h1p, dinv, b1.reshape(1, D), W2)

    agg2 = _from_feature_major(_agg_kernel(_to_feature_major(h2p), ei_agg, w_agg))

    out = pl.pallas_call(
        _tc3_body,
        out_shape=jax.ShapeDtypeStruct((N, D), _f32),
    )(agg2, h2p, dinv, b2.reshape(1, D), bn_g.reshape(1, D),
      bn_b.reshape(1, D), ln_g.reshape(1, D), ln_b.reshape(1, D))
    return out


# speedup vs baseline: 1.3389x; 1.3389x over previous
import dataclasses
import functools

import numpy as np
import jax
import jax.numpy as jnp
from jax import lax
from jax.experimental import pallas as pl
from jax.experimental.pallas import tpu as pltpu
from jax.experimental.pallas import tpu_sc as plsc

N = 10000
E = 320000
D = 128
NC, NS = 2, 16
NW = NC * NS
BLK = 128
NB = 79
E_PAD = NW * NB * BLK
N_PAD = 10240
RPT = N_PAD // NS
LANES = 16

_PERM = np.arange(D).reshape(D // 32, 2, 16).transpose(0, 2, 1).reshape(D)

_mesh = plsc.VectorSubcoreMesh(
    core_axis_name="c", subcore_axis_name="s", num_cores=NC, num_subcores=NS
)

_f32 = jnp.float32

_sc_params = pltpu.CompilerParams()
if "needs_layout_passes" in pltpu.CompilerParams.__dataclass_fields__:
    _sc_params = dataclasses.replace(_sc_params, needs_layout_passes=False)
_sc_params_linear = dataclasses.replace(_sc_params, use_tc_tiling_on_sc=False)


@functools.partial(
    pl.kernel,
    out_type=jax.ShapeDtypeStruct((NW * N,), _f32),
    mesh=_mesh,
    scratch_types=[
        pltpu.VMEM((NB, BLK), jnp.int32),
        pltpu.VMEM((NB, BLK), _f32),
        pltpu.VMEM((N,), _f32),
    ],
    compiler_params=_sc_params,
)
def _deg_kernel(ei_hbm, w_hbm, out_hbm, colb, wb, degloc):
    c = lax.axis_index("c")
    s = lax.axis_index("s")
    wid = c * NS + s
    pltpu.sync_copy(ei_hbm.at[1, wid], colb)
    pltpu.sync_copy(w_hbm.at[wid], wb)

    @pl.loop(0, N, step=LANES)
    def _zero(i):
        degloc[pl.ds(i, LANES)] = jnp.zeros((LANES,), _f32)

    @pl.loop(0, NB)
    def _blocks(blk):
        @pl.loop(0, BLK, step=LANES)
        def _groups(j0):
            col16 = colb[blk, pl.ds(j0, LANES)]
            w16 = wb[blk, pl.ds(j0, LANES)]
            plsc.addupdate_scatter(degloc, [col16], w16)

    pltpu.sync_copy(degloc, out_hbm.at[pl.ds(wid * N, N)])


@functools.partial(
    pl.kernel,
    out_type=jax.ShapeDtypeStruct((NC, N_PAD, D), _f32),
    mesh=_mesh,
    scratch_types=[
        pltpu.VMEM((NB, BLK), jnp.int32),
        pltpu.VMEM((NB, BLK), jnp.int32),
        pltpu.VMEM((BLK,), _f32),
        pltpu.VMEM((BLK, D), jnp.bfloat16),
        pltpu.VMEM((BLK, D), _f32),
        pltpu.VMEM_SHARED((N_PAD, D), _f32),
        pltpu.SemaphoreType.DMA,
    ],
    compiler_params=_sc_params_linear,
)
def _agg_kernel(hb_hbm, ei_hbm, w_hbm, out_hbm,
                rowb, colb, wstage, bufh, buff, acc, wsem):
    c = lax.axis_index("c")
    s = lax.axis_index("s")
    wid = c * NS + s
    pltpu.sync_copy(ei_hbm.at[0, wid], rowb)
    pltpu.sync_copy(ei_hbm.at[1, wid], colb)

    @pl.loop(0, BLK)
    def _zrow(i):
        for k in range(D // LANES):
            buff[i, pl.ds(k * LANES, LANES)] = jnp.zeros((LANES,), _f32)

    for r in range(RPT // BLK):
        pltpu.sync_copy(buff, acc.at[pl.ds(s * RPT + r * BLK, BLK)])
    plsc.subcore_barrier()

    @pl.loop(0, NB)
    def _blocks(blk):
        pltpu.async_copy(w_hbm.at[wid, blk], wstage, wsem)
        pltpu.sync_copy(hb_hbm.at[rowb.at[blk]], bufh)
        pltpu.make_async_copy(w_hbm.at[wid, blk], wstage, wsem).wait()

        @pl.loop(0, BLK, step=LANES)
        def _groups(j0):
            w16 = wstage[pl.ds(j0, LANES)]
            for jj in range(LANES):
                sp = w16.at[jnp.full((LANES,), jj, jnp.int32)].get(
                    mode="promise_in_bounds"
                )
                for k in range(D // 32):
                    pair = bufh[j0 + jj, pl.ds(k * 32, 32)]
                    lo, hi = plsc.unpack(pair, format=plsc.PackFormat.INTERLEAVED)
                    buff[j0 + jj, pl.ds(k * 32, LANES)] = lo * sp
                    buff[j0 + jj, pl.ds(k * 32 + LANES, LANES)] = hi * sp

        pltpu.sync_copy(buff, acc.at[colb.at[blk]], add=True)

    plsc.subcore_barrier()
    pltpu.sync_copy(acc.at[pl.ds(s * RPT, RPT)], out_hbm.at[c, pl.ds(s * RPT, RPT)])


def _tc1_body(parts_ref, x_ref, w1_ref, hb_ref, h1p_ref, dinv_ref):
    deg = jnp.sum(parts_ref[...], axis=0) + 1.0
    dinv = jnp.where(deg > 0, lax.rsqrt(deg), 0.0)[:, None]
    h1 = jnp.dot(x_ref[...], w1_ref[...], preferred_element_type=_f32)
    h1p = h1 * dinv
    h1p_ref[...] = h1p
    dinv_ref[...] = dinv
    hb_ref[...] = h1p.astype(jnp.bfloat16)


def _tc2_body(acc_ref, h1p_ref, dinv_ref, b1_ref, w2_ref, hb_ref, h2p_ref):
    dinv = dinv_ref[...]
    agg = acc_ref[0, :N] + acc_ref[1, :N]
    x2 = jnp.maximum((agg + h1p_ref[...]) * dinv + b1_ref[...], 0.0)
    h2 = jnp.dot(x2, w2_ref[...], preferred_element_type=_f32)
    h2p = h2 * dinv
    h2p_ref[...] = h2p
    hb_ref[...] = h2p.astype(jnp.bfloat16)


def _tc3_body(acc_ref, h2p_ref, dinv_ref, b2_ref, bn_g_ref, bn_b_ref,
              ln_g_ref, ln_b_ref, out_ref):
    dinv = dinv_ref[...]
    agg = acc_ref[0, :N] + acc_ref[1, :N]
    t = jnp.maximum((agg + h2p_ref[...]) * dinv + b2_ref[...], 0.0)
    mu = jnp.mean(t, axis=0, keepdims=True)
    var = jnp.mean((t - mu) ** 2, axis=0, keepdims=True)
    h = (t - mu) / jnp.sqrt(var + 1e-5) * bn_g_ref[...] + bn_b_ref[...]
    lmu = jnp.mean(h, axis=1, keepdims=True)
    lvar = jnp.mean((h - lmu) ** 2, axis=1, keepdims=True)
    out_ref[...] = (h - lmu) / jnp.sqrt(lvar + 1e-5) * ln_g_ref[...] + ln_b_ref[...]


def kernel(x, edge_index, edge_weight, W1, b1, W2, b2, bn_g, bn_b, ln_g, ln_b):
    pad = E_PAD - E
    ei3 = jnp.concatenate(
        [edge_index, jnp.zeros((2, pad), edge_index.dtype)], axis=1
    ).reshape(2, NW, NB, BLK)
    w3 = jnp.concatenate(
        [edge_weight, jnp.zeros((pad,), edge_weight.dtype)]
    ).reshape(NW, NB, BLK)

    parts = _deg_kernel(ei3, w3).reshape(NW, N)

    hb1, h1p, dinv = pl.pallas_call(
        _tc1_body,
        out_shape=[jax.ShapeDtypeStruct((N, D), jnp.bfloat16),
                   jax.ShapeDtypeStruct((N, D), _f32),
                   jax.ShapeDtypeStruct((N, 1), _f32)],
    )(parts, x, W1)

    acc1 = _agg_kernel(hb1[:, _PERM], ei3, w3)

    hb2, h2p = pl.pallas_call(
        _tc2_body,
        out_shape=[jax.ShapeDtypeStruct((N, D), jnp.bfloat16),
                   jax.ShapeDtypeStruct((N, D), _f32)],
    )(acc1, h1p, dinv, b1.reshape(1, D), W2)

    acc2 = _agg_kernel(hb2[:, _PERM], ei3, w3)

    out = pl.pallas_call(
        _tc3_body,
        out_shape=jax.ShapeDtypeStruct((N, D), _f32),
    )(acc2, h2p, dinv, b2.reshape(1, D), bn_g.reshape(1, D),
      bn_b.reshape(1, D), ln_g.reshape(1, D), ln_b.reshape(1, D))
    return out


# speedup vs baseline: 1.5865x; 1.1849x over previous
import dataclasses
import functools

import jax
import jax.numpy as jnp
from jax import lax
from jax.experimental import pallas as pl
from jax.experimental.pallas import tpu as pltpu
from jax.experimental.pallas import tpu_sc as plsc

N = 10000
E = 320000
D = 128
NC, NS = 2, 16
NW = NC * NS
BLK = 128
NB = 79
E_PAD = NW * NB * BLK
N_PAD = 10240
RPT = N_PAD // NS
LANES = 16

_mesh = plsc.VectorSubcoreMesh(
    core_axis_name="c", subcore_axis_name="s", num_cores=NC, num_subcores=NS
)

_f32 = jnp.float32

_sc_params = pltpu.CompilerParams()
if "needs_layout_passes" in pltpu.CompilerParams.__dataclass_fields__:
    _sc_params = dataclasses.replace(_sc_params, needs_layout_passes=False)


@functools.partial(
    pl.kernel,
    out_type=jax.ShapeDtypeStruct((NW * N,), _f32),
    mesh=_mesh,
    scratch_types=[
        pltpu.VMEM((NB, BLK), jnp.int32),
        pltpu.VMEM((NB, BLK), _f32),
        pltpu.VMEM((N,), _f32),
    ],
    compiler_params=_sc_params,
)
def _deg_kernel(ei_hbm, w_hbm, out_hbm, colb, wb, degloc):
    c = lax.axis_index("c")
    s = lax.axis_index("s")
    wid = c * NS + s
    pltpu.sync_copy(ei_hbm.at[1, wid], colb)
    pltpu.sync_copy(w_hbm.at[wid], wb)

    @pl.loop(0, N, step=LANES)
    def _zero(i):
        degloc[pl.ds(i, LANES)] = jnp.zeros((LANES,), _f32)

    @pl.loop(0, NB)
    def _blocks(blk):
        @pl.loop(0, BLK, step=LANES)
        def _groups(j0):
            col16 = colb[blk, pl.ds(j0, LANES)]
            w16 = wb[blk, pl.ds(j0, LANES)]
            plsc.addupdate_scatter(degloc, [col16], w16)

    pltpu.sync_copy(degloc, out_hbm.at[pl.ds(wid * N, N)])


@functools.partial(
    pl.kernel,
    out_type=jax.ShapeDtypeStruct((NC, N_PAD, D), _f32),
    mesh=_mesh,
    scratch_types=[
        pltpu.VMEM((NB, BLK), jnp.int32),
        pltpu.VMEM((NB, BLK), jnp.int32),
        pltpu.VMEM((BLK,), _f32),
        pltpu.VMEM((BLK, D), _f32),
        pltpu.VMEM_SHARED((N_PAD, D), _f32),
        pltpu.SemaphoreType.DMA,
    ],
    compiler_params=_sc_params,
)
def _agg_kernel(h_hbm, ei_hbm, w_hbm, out_hbm,
                rowb, colb, wstage, buf, acc, wsem):
    c = lax.axis_index("c")
    s = lax.axis_index("s")
    wid = c * NS + s
    pltpu.sync_copy(ei_hbm.at[0, wid], rowb)
    pltpu.sync_copy(ei_hbm.at[1, wid], colb)

    @pl.loop(0, BLK)
    def _zrow(i):
        for k in range(D // LANES):
            buf[i, pl.ds(k * LANES, LANES)] = jnp.zeros((LANES,), _f32)

    for r in range(RPT // BLK):
        pltpu.sync_copy(buf, acc.at[pl.ds(s * RPT + r * BLK, BLK)])
    plsc.subcore_barrier()

    @pl.loop(0, NB)
    def _blocks(blk):
        pltpu.async_copy(w_hbm.at[wid, blk], wstage, wsem)
        pltpu.sync_copy(h_hbm.at[rowb.at[blk]], buf)
        pltpu.make_async_copy(w_hbm.at[wid, blk], wstage, wsem).wait()

        @pl.loop(0, BLK, step=LANES)
        def _groups(j0):
            w16 = wstage[pl.ds(j0, LANES)]
            for jj in range(LANES):
                sp = w16.at[jnp.full((LANES,), jj, jnp.int32)].get(
                    mode="promise_in_bounds"
                )
                for k in range(D // LANES):
                    sl = pl.ds(k * LANES, LANES)
                    buf[j0 + jj, sl] = buf[j0 + jj, sl] * sp

        pltpu.sync_copy(buf, acc.at[colb.at[blk]], add=True)

    plsc.subcore_barrier()
    pltpu.sync_copy(acc.at[pl.ds(s * RPT, RPT)], out_hbm.at[c, pl.ds(s * RPT, RPT)])


def _tc1_body(parts_ref, x_ref, w1_ref, h1p_ref, dinv_ref):
    deg = jnp.sum(parts_ref[...], axis=0) + 1.0
    dinv = jnp.where(deg > 0, lax.rsqrt(deg), 0.0)[:, None]
    h1 = jnp.dot(x_ref[...], w1_ref[...], preferred_element_type=_f32)
    h1p_ref[...] = h1 * dinv
    dinv_ref[...] = dinv


def _tc2_body(acc_ref, h1p_ref, dinv_ref, b1_ref, w2_ref, h2p_ref):
    dinv = dinv_ref[...]
    agg = acc_ref[0, :N] + acc_ref[1, :N]
    x2 = jnp.maximum((agg + h1p_ref[...]) * dinv + b1_ref[...], 0.0)
    h2 = jnp.dot(x2, w2_ref[...], preferred_element_type=_f32)
    h2p_ref[...] = h2 * dinv


def _tc3_body(acc_ref, h2p_ref, dinv_ref, b2_ref, bn_g_ref, bn_b_ref,
              ln_g_ref, ln_b_ref, out_ref):
    dinv = dinv_ref[...]
    agg = acc_ref[0, :N] + acc_ref[1, :N]
    t = jnp.maximum((agg + h2p_ref[...]) * dinv + b2_ref[...], 0.0)
    mu = jnp.mean(t, axis=0, keepdims=True)
    var = jnp.mean((t - mu) ** 2, axis=0, keepdims=True)
    h = (t - mu) / jnp.sqrt(var + 1e-5) * bn_g_ref[...] + bn_b_ref[...]
    lmu = jnp.mean(h, axis=1, keepdims=True)
    lvar = jnp.mean((h - lmu) ** 2, axis=1, keepdims=True)
    out_ref[...] = (h - lmu) / jnp.sqrt(lvar + 1e-5) * ln_g_ref[...] + ln_b_ref[...]


def kernel(x, edge_index, edge_weight, W1, b1, W2, b2, bn_g, bn_b, ln_g, ln_b):
    pad = E_PAD - E
    ei3 = jnp.concatenate(
        [edge_index, jnp.zeros((2, pad), edge_index.dtype)], axis=1
    ).reshape(2, NW, NB, BLK)
    w3 = jnp.concatenate(
        [edge_weight, jnp.zeros((pad,), edge_weight.dtype)]
    ).reshape(NW, NB, BLK)

    parts = _deg_kernel(ei3, w3).reshape(NW, N)

    h1p, dinv = pl.pallas_call(
        _tc1_body,
        out_shape=[jax.ShapeDtypeStruct((N, D), _f32),
                   jax.ShapeDtypeStruct((N, 1), _f32)],
    )(parts, x, W1)

    acc1 = _agg_kernel(h1p, ei3, w3)

    h2p = pl.pallas_call(
        _tc2_body,
        out_shape=jax.ShapeDtypeStruct((N, D), _f32),
    )(acc1, h1p, dinv, b1.reshape(1, D), W2)

    acc2 = _agg_kernel(h2p, ei3, w3)

    out = pl.pallas_call(
        _tc3_body,
        out_shape=jax.ShapeDtypeStruct((N, D), _f32),
    )(acc2, h2p, dinv, b2.reshape(1, D), bn_g.reshape(1, D),
      bn_b.reshape(1, D), ln_g.reshape(1, D), ln_b.reshape(1, D))
    return out
